# baseline (speedup 1.0000x reference)
import functools

import jax
import jax.numpy as jnp
from jax import lax
from jax.experimental import pallas as pl
from jax.experimental.pallas import tpu as pltpu
from jax.experimental.pallas import tpu_sc as plsc

NC = 2
NS = 16
NW = NC * NS
CB = 8
NSET = 4
HP = 56
EP = 128


@functools.lru_cache(maxsize=None)
def _densify(vocab: int, embed: int):
    rows = 800
    grid = vocab // rows

    def body(i_ref, o_ref):
        x = i_ref[...].reshape(rows // 2, 2, embed)
        o_ref[...] = jnp.concatenate([x[:, 0, :], x[:, 1, :]], axis=1)

    return pl.pallas_call(
        body,
        grid=(grid,),
        in_specs=[pl.BlockSpec((rows, embed), lambda i: (i, 0))],
        out_specs=pl.BlockSpec((rows // 2, 2 * embed), lambda i: (i, 0)),
        out_shape=jax.ShapeDtypeStruct((vocab // 2, 2 * embed), jnp.float32),
    )


@functools.lru_cache(maxsize=None)
def _build(batch: int, hist: int, vocab: int, embed: int):
    assert batch % (NW * CB) == 0
    rows_per_w = batch // NW
    n_chunks = rows_per_w // CB
    assert n_chunks > NSET
    mesh = plsc.VectorSubcoreMesh(core_axis_name="c", subcore_axis_name="s")

    @functools.partial(
        pl.kernel,
        mesh=mesh,
        compiler_params=pltpu.CompilerParams(use_tc_tiling_on_sc=False),
        out_type=jax.ShapeDtypeStruct((NW, n_chunks, CB, HP, EP), jnp.float32),
        scratch_types=[
            pltpu.VMEM((rows_per_w, hist), jnp.int32),
            pltpu.VMEM((NSET, CB, hist, embed), jnp.float32),
            pltpu.SemaphoreType.DMA((NSET,)),
            pltpu.SemaphoreType.DMA,
        ],
    )
    def emb_kernel(idx_hbm, table_hbm, out_hbm, idx_v, stage_v, sem_g, sem_o):
        wid = lax.axis_index("s") * NC + lax.axis_index("c")
        pltpu.sync_copy(idx_hbm.at[wid], idx_v)

        def fire_gathers(c):
            s = lax.rem(c, NSET)
            for bb in range(CB):
                pltpu.async_copy(
                    table_hbm.at[idx_v.at[c * CB + bb]],
                    stage_v.at[s, bb],
                    sem_g.at[s],
                )

        def fire_write(c):
            s = lax.rem(c, NSET)
            pltpu.async_copy(
                stage_v.at[s],
                out_hbm.at[wid, c, slice(None), pl.ds(0, hist), pl.ds(0, embed)],
                sem_o,
            )

        def drain_g(c, n):
            s = lax.rem(c, NSET)
            for _ in range(n):
                pltpu.make_async_copy(
                    out_hbm.at[wid, 0, 0, pl.ds(0, hist), pl.ds(0, embed)],
                    stage_v.at[0, 0],
                    sem_g.at[s],
                ).wait()

        def drain_o(n):
            for _ in range(n):
                pltpu.make_async_copy(
                    stage_v.at[0],
                    out_hbm.at[wid, 0, slice(None), pl.ds(0, hist), pl.ds(0, embed)],
                    sem_o,
                ).wait()

        for c in range(NSET - 1):
            fire_gathers(c)

        drain_g(0, CB)
        fire_write(0)
        fire_gathers(NSET - 1)

        @pl.loop(1, n_chunks - (NSET - 1))
        def _(c):
            drain_g(c, CB)
            drain_o(1)
            fire_write(c)
            fire_gathers(c + NSET - 1)

        @pl.loop(n_chunks - (NSET - 1), n_chunks)
        def _(c):
            drain_g(c, CB)
            fire_write(c)

        drain_o(NSET)

    return emb_kernel


def kernel(x, vectors):
    batch, hist = x.shape
    vocab, embed = vectors.shape
    idx = x.astype(jnp.int32).reshape(NW, batch // NW, hist)
    tbl = _densify(vocab, embed)(vectors).reshape(vocab, embed)
    out = _build(batch, hist, vocab, embed)(idx, tbl)
    return out.reshape(batch, HP, EP)[:, :hist, :embed]


# speedup vs baseline: 1.4189x; 1.4189x over previous
import functools

import jax
import jax.numpy as jnp
from jax import lax
from jax.experimental import pallas as pl
from jax.experimental.pallas import tpu as pltpu
from jax.experimental.pallas import tpu_sc as plsc

NC = 2
NS = 16
NW = NC * NS
CB = 8
NSET = 4
HP = 56
EP = 128


@functools.lru_cache(maxsize=None)
def _build(batch: int, hist: int, vocab: int, embed: int):
    assert batch % (NW * CB) == 0
    rows_per_w = batch // NW
    n_chunks = rows_per_w // CB
    assert n_chunks > NSET
    mesh = plsc.VectorSubcoreMesh(core_axis_name="c", subcore_axis_name="s")

    @functools.partial(
        pl.kernel,
        mesh=mesh,
        compiler_params=pltpu.CompilerParams(use_tc_tiling_on_sc=False),
        out_type=jax.ShapeDtypeStruct((NW, n_chunks, CB, HP, EP), jnp.float32),
        scratch_types=[
            pltpu.VMEM((rows_per_w, hist), jnp.int32),
            pltpu.VMEM((NSET, CB, hist, embed), jnp.float32),
            pltpu.SemaphoreType.DMA((NSET,)),
            pltpu.SemaphoreType.DMA,
        ],
    )
    def emb_kernel(idx_hbm, table_hbm, out_hbm, idx_v, stage_v, sem_g, sem_o):
        wid = lax.axis_index("s") * NC + lax.axis_index("c")
        pltpu.sync_copy(idx_hbm.at[wid], idx_v)

        def fire_gathers(c):
            s = lax.rem(c, NSET)
            for bb in range(CB):
                pltpu.async_copy(
                    table_hbm.at[idx_v.at[c * CB + bb]],
                    stage_v.at[s, bb],
                    sem_g.at[s],
                )

        def fire_write(c):
            s = lax.rem(c, NSET)
            pltpu.async_copy(
                stage_v.at[s],
                out_hbm.at[wid, c, slice(None), pl.ds(0, hist), pl.ds(0, embed)],
                sem_o,
            )

        def drain_g(c, n):
            s = lax.rem(c, NSET)
            for _ in range(n):
                pltpu.make_async_copy(
                    out_hbm.at[wid, 0, 0, pl.ds(0, hist), pl.ds(0, embed)],
                    stage_v.at[0, 0],
                    sem_g.at[s],
                ).wait()

        def drain_o(n):
            for _ in range(n):
                pltpu.make_async_copy(
                    stage_v.at[0],
                    out_hbm.at[wid, 0, slice(None), pl.ds(0, hist), pl.ds(0, embed)],
                    sem_o,
                ).wait()

        for c in range(NSET - 1):
            fire_gathers(c)

        drain_g(0, CB)
        fire_write(0)
        fire_gathers(NSET - 1)

        @pl.loop(1, n_chunks - (NSET - 1))
        def _(c):
            drain_g(c, CB)
            drain_o(1)
            fire_write(c)
            fire_gathers(c + NSET - 1)

        @pl.loop(n_chunks - (NSET - 1), n_chunks)
        def _(c):
            drain_g(c, CB)
            fire_write(c)

        drain_o(NSET)

    return emb_kernel


def kernel(x, vectors):
    batch, hist = x.shape
    vocab, embed = vectors.shape
    idx = x.astype(jnp.int32).reshape(NW, batch // NW, hist)
    out = _build(batch, hist, vocab, embed)(idx, vectors)
    return out.reshape(batch, HP, EP)[:, :hist, :embed]
